# baseline (speedup 1.0000x reference)
import functools

import jax
import jax.numpy as jnp
from jax import lax
from jax.experimental import pallas as pl
from jax.experimental.pallas import tpu as pltpu
from jax.experimental.pallas import tpu_sc as plsc

N_ROWS = 16384
N_COLS = 832
N_FIELDS_K = 26
FIELD = 32
LANES = 16
IDX_COLS = 32

NW = 32
ROWS_PER_W = N_ROWS // NW
CHUNK = 32
N_CHUNKS = ROWS_PER_W // CHUNK

TC_BLOCK = 512


def _sc_body(x_hbm, out_hbm, in_a, in_b, out_a, out_b,
             s_ia, s_ib, s_oa, s_ob):
    wid = lax.axis_index("s") * 2 + lax.axis_index("c")
    ji = lax.iota(jnp.int32, LANES)
    lane0 = ji == 0
    row0 = wid * ROWS_PER_W

    def in_copy(k, buf, sem):
        return pltpu.make_async_copy(
            x_hbm.at[pl.ds(row0 + k * CHUNK, CHUNK)], buf, sem)

    def out_copy(k, buf, sem):
        return pltpu.make_async_copy(
            buf, out_hbm.at[pl.ds((row0 + k * CHUNK) * IDX_COLS,
                                  CHUNK * IDX_COLS)], sem)

    def compute(ibuf, obuf):
        def row_body(r, _):
            rb = r * IDX_COLS
            for f in range(N_FIELDS_K):
                c = f * FIELD
                v0 = ibuf[r, pl.ds(c, LANES)]
                v1 = ibuf[r, pl.ds(c + LANES, LANES)]
                m = jnp.max(jnp.maximum(v0, v1))
                f0 = plsc.all_reduce_ffs(v0 == m)
                f1 = plsc.all_reduce_ffs(v1 == m)
                first = jnp.where(f0 < LANES, f0, f1 + LANES)
                plsc.store_scatter(obuf, [ji + (rb + f)], first,
                                   mask=lane0)
            return 0

        lax.fori_loop(0, CHUNK, row_body, 0)

    in_copy(0, in_a, s_ia).start()
    in_copy(1, in_b, s_ib).start()

    in_copy(0, in_a, s_ia).wait()
    compute(in_a, out_a)
    out_copy(0, out_a, s_oa).start()
    in_copy(2, in_a, s_ia).start()

    in_copy(1, in_b, s_ib).wait()
    compute(in_b, out_b)
    out_copy(1, out_b, s_ob).start()
    in_copy(3, in_b, s_ib).start()

    def pair_body(jj, _):
        k0 = 2 * jj
        k1 = k0 + 1
        in_copy(k0, in_a, s_ia).wait()
        out_copy(k0 - 2, out_a, s_oa).wait()
        compute(in_a, out_a)
        out_copy(k0, out_a, s_oa).start()
        in_copy(k0 + 2, in_a, s_ia).start()

        in_copy(k1, in_b, s_ib).wait()
        out_copy(k1 - 2, out_b, s_ob).wait()
        compute(in_b, out_b)
        out_copy(k1, out_b, s_ob).start()
        in_copy(k1 + 2, in_b, s_ib).start()
        return 0

    lax.fori_loop(1, N_CHUNKS // 2 - 1, pair_body, 0)

    kl = N_CHUNKS - 2
    in_copy(kl, in_a, s_ia).wait()
    out_copy(kl - 2, out_a, s_oa).wait()
    compute(in_a, out_a)
    out_copy(kl, out_a, s_oa).start()

    in_copy(kl + 1, in_b, s_ib).wait()
    out_copy(kl - 1, out_b, s_ob).wait()
    compute(in_b, out_b)
    out_copy(kl + 1, out_b, s_ob).start()

    out_copy(kl, out_a, s_oa).wait()
    out_copy(kl + 1, out_b, s_ob).wait()


def _tc_expand_body(idx_ref, out_ref):
    idxb = idx_ref[...]
    for f in range(N_FIELDS_K):
        col = idxb[:, f:f + 1]
        pos = lax.broadcasted_iota(jnp.int32, (1, FIELD), 1)
        out_ref[:, pl.ds(f * FIELD, FIELD)] = jnp.where(
            col == pos, 1.0, 0.0).astype(jnp.float32)


@jax.jit
def kernel(x):
    mesh = plsc.VectorSubcoreMesh(core_axis_name="c", subcore_axis_name="s")
    sc = functools.partial(
        pl.kernel,
        mesh=mesh,
        out_type=jax.ShapeDtypeStruct((N_ROWS * IDX_COLS,), jnp.int32),
        scratch_types=[
            pltpu.VMEM((CHUNK, N_COLS), jnp.float32),
            pltpu.VMEM((CHUNK, N_COLS), jnp.float32),
            pltpu.VMEM((CHUNK * IDX_COLS,), jnp.int32),
            pltpu.VMEM((CHUNK * IDX_COLS,), jnp.int32),
            pltpu.SemaphoreType.DMA,
            pltpu.SemaphoreType.DMA,
            pltpu.SemaphoreType.DMA,
            pltpu.SemaphoreType.DMA,
        ],
        compiler_params=pltpu.CompilerParams(needs_layout_passes=False),
    )(_sc_body)
    idx = sc(x).reshape(N_ROWS, IDX_COLS)

    expand = pl.pallas_call(
        _tc_expand_body,
        grid=(N_ROWS // TC_BLOCK,),
        in_specs=[pl.BlockSpec((TC_BLOCK, IDX_COLS), lambda i: (i, 0))],
        out_specs=pl.BlockSpec((TC_BLOCK, N_COLS), lambda i: (i, 0)),
        out_shape=jax.ShapeDtypeStruct((N_ROWS, N_COLS), jnp.float32),
    )
    return expand(idx)


# speedup vs baseline: 2.2118x; 2.2118x over previous
import functools

import jax
import jax.numpy as jnp
from jax import lax
from jax.experimental import pallas as pl
from jax.experimental.pallas import tpu as pltpu
from jax.experimental.pallas import tpu_sc as plsc

N_ROWS = 16384
N_COLS = 832
N_FIELDS_K = 26
FIELD = 32
LANES = 16

NW = 32
ROWS_PER_W = N_ROWS // NW
CHUNK = 32
N_CHUNKS = ROWS_PER_W // CHUNK


def _sc_body(x_hbm, out_hbm, in_a, in_b, out_a, out_b,
             s_ia, s_ib, s_oa, s_ob):
    wid = lax.axis_index("s") * 2 + lax.axis_index("c")
    ji = lax.iota(jnp.int32, LANES)
    ji16 = ji + LANES
    row0 = wid * ROWS_PER_W

    def in_copy(k, buf, sem):
        return pltpu.make_async_copy(
            x_hbm.at[pl.ds(row0 + k * CHUNK, CHUNK)], buf, sem)

    def out_copy(k, buf, sem):
        return pltpu.make_async_copy(
            buf, out_hbm.at[pl.ds(row0 + k * CHUNK, CHUNK)], sem)

    def compute(ibuf, obuf):
        def one_seg(r, c):
            v0 = ibuf[r, pl.ds(c, LANES)]
            v1 = ibuf[r, pl.ds(c + LANES, LANES)]
            m = jnp.max(jnp.maximum(v0, v1))
            f0 = plsc.all_reduce_ffs(v0 == m)
            f1 = plsc.all_reduce_ffs(v1 == m)
            first = jnp.where(f0 < LANES, f0, f1 + LANES)
            obuf[r, pl.ds(c, LANES)] = jnp.where(
                ji == first, 1.0, 0.0).astype(jnp.float32)
            obuf[r, pl.ds(c + LANES, LANES)] = jnp.where(
                ji16 == first, 1.0, 0.0).astype(jnp.float32)

        @plsc.parallel_loop(0, CHUNK, unroll=2)
        def row_body(r):
            for f in range(N_FIELDS_K):
                one_seg(r, f * FIELD)

    in_copy(0, in_a, s_ia).start()
    in_copy(1, in_b, s_ib).start()

    in_copy(0, in_a, s_ia).wait()
    compute(in_a, out_a)
    out_copy(0, out_a, s_oa).start()
    in_copy(2, in_a, s_ia).start()

    in_copy(1, in_b, s_ib).wait()
    compute(in_b, out_b)
    out_copy(1, out_b, s_ob).start()
    in_copy(3, in_b, s_ib).start()

    def pair_body(jj, _):
        k0 = 2 * jj
        k1 = k0 + 1
        in_copy(k0, in_a, s_ia).wait()
        out_copy(k0 - 2, out_a, s_oa).wait()
        compute(in_a, out_a)
        out_copy(k0, out_a, s_oa).start()
        in_copy(k0 + 2, in_a, s_ia).start()

        in_copy(k1, in_b, s_ib).wait()
        out_copy(k1 - 2, out_b, s_ob).wait()
        compute(in_b, out_b)
        out_copy(k1, out_b, s_ob).start()
        in_copy(k1 + 2, in_b, s_ib).start()
        return 0

    lax.fori_loop(1, N_CHUNKS // 2 - 1, pair_body, 0)

    kl = N_CHUNKS - 2
    in_copy(kl, in_a, s_ia).wait()
    out_copy(kl - 2, out_a, s_oa).wait()
    compute(in_a, out_a)
    out_copy(kl, out_a, s_oa).start()

    in_copy(kl + 1, in_b, s_ib).wait()
    out_copy(kl - 1, out_b, s_ob).wait()
    compute(in_b, out_b)
    out_copy(kl + 1, out_b, s_ob).start()

    out_copy(kl, out_a, s_oa).wait()
    out_copy(kl + 1, out_b, s_ob).wait()


@jax.jit
def kernel(x):
    mesh = plsc.VectorSubcoreMesh(core_axis_name="c", subcore_axis_name="s")
    f = functools.partial(
        pl.kernel,
        mesh=mesh,
        out_type=jax.ShapeDtypeStruct((N_ROWS, N_COLS), jnp.float32),
        scratch_types=[
            pltpu.VMEM((CHUNK, N_COLS), jnp.float32),
            pltpu.VMEM((CHUNK, N_COLS), jnp.float32),
            pltpu.VMEM((CHUNK, N_COLS), jnp.float32),
            pltpu.VMEM((CHUNK, N_COLS), jnp.float32),
            pltpu.SemaphoreType.DMA,
            pltpu.SemaphoreType.DMA,
            pltpu.SemaphoreType.DMA,
            pltpu.SemaphoreType.DMA,
        ],
        compiler_params=pltpu.CompilerParams(needs_layout_passes=False),
    )(_sc_body)
    return f(x)
